# baseline (speedup 1.0000x reference)
import functools

import jax
import jax.numpy as jnp
from jax import lax
from jax.experimental import pallas as pl
from jax.experimental.pallas import tpu as pltpu
from jax.experimental.pallas import tpu_sc as plsc

_NC = 2
_NS = 16
_CH = 64


def _tc_proj(x, W, b, blk):
    M, K = x.shape
    Do = W.shape[1]

    def body(x_ref, w_ref, b_ref, o_ref):
        v = jnp.dot(x_ref[...], w_ref[...], preferred_element_type=jnp.float32)
        o_ref[...] = jnp.maximum(v + b_ref[...], 0.0)

    return pl.pallas_call(
        body,
        grid=(M // blk,),
        in_specs=[
            pl.BlockSpec((blk, K), lambda i: (i, 0)),
            pl.BlockSpec((K, Do), lambda i: (0, 0)),
            pl.BlockSpec((1, Do), lambda i: (0, 0)),
        ],
        out_specs=pl.BlockSpec((blk, Do), lambda i: (i, 0)),
        out_shape=jax.ShapeDtypeStruct((M, Do), jnp.float32),
    )(x, W, b.reshape(1, Do))


def _tc_edge_proj(ea, W, b, E_pad, blk):
    M, K = ea.shape
    Do = W.shape[1]
    last = (M - 1) // blk

    def body(a_ref, w_ref, b_ref, o_ref):
        v = jnp.dot(a_ref[...], w_ref[...], preferred_element_type=jnp.float32)
        v = jnp.maximum(v + b_ref[...], 0.0)
        bits = lax.bitcast_convert_type(v, jnp.int32)
        rnd = bits + jnp.int32(0x7FFF) + ((bits >> 16) & 1)
        lo = (rnd[:, : Do // 2] >> 16) & jnp.int32(0xFFFF)
        hi = rnd[:, Do // 2 :] & jnp.int32(-65536)
        o_ref[...] = lo | hi

    return pl.pallas_call(
        body,
        grid=(E_pad // blk,),
        in_specs=[
            pl.BlockSpec((blk, K), lambda i: (jnp.minimum(i, last), 0)),
            pl.BlockSpec((K, Do), lambda i: (0, 0)),
            pl.BlockSpec((1, Do), lambda i: (0, 0)),
        ],
        out_specs=pl.BlockSpec((blk, Do // 2), lambda i: (i, 0)),
        out_shape=jax.ShapeDtypeStruct((E_pad, Do // 2), jnp.int32),
    )(ea, W, b.reshape(1, Do))


def _tc_update(h, a0, a1, W, b, blk):
    M, K = h.shape
    Do = W.shape[1]

    def body(h_ref, a0_ref, a1_ref, w_ref, b_ref, o_ref):
        t = h_ref[...] + a0_ref[...] + a1_ref[...]
        v = jnp.dot(t, w_ref[...], preferred_element_type=jnp.float32)
        o_ref[...] = jnp.maximum(v + b_ref[...], 0.0)

    return pl.pallas_call(
        body,
        grid=(M // blk,),
        in_specs=[
            pl.BlockSpec((blk, K), lambda i: (i, 0)),
            pl.BlockSpec((blk, K), lambda i: (i, 0)),
            pl.BlockSpec((blk, K), lambda i: (i, 0)),
            pl.BlockSpec((K, Do), lambda i: (0, 0)),
            pl.BlockSpec((1, Do), lambda i: (0, 0)),
        ],
        out_specs=pl.BlockSpec((blk, Do), lambda i: (i, 0)),
        out_shape=jax.ShapeDtypeStruct((M, Do), jnp.float32),
    )(h, a0, a1, W, b.reshape(1, Do))


@functools.cache
def _make_sc_layer(N, D, E_pad, N_pad, C0, C1):
    assert (C0 + C1) * _NS * _CH == E_pad and C0 % 4 == 0 and C1 % 4 == 0
    RPT = N_pad // _NS
    ZR = (RPT + _CH - 1) // _CH
    mesh = plsc.VectorSubcoreMesh(core_axis_name="c", subcore_axis_name="s")

    @functools.partial(
        pl.kernel,
        out_type=jax.ShapeDtypeStruct((_NC * N_pad, D), jnp.float32),
        mesh=mesh,
        scratch_types=[
            pltpu.VMEM((4, _CH), jnp.int32),
            pltpu.VMEM((4, _CH), jnp.int32),
            pltpu.VMEM((2, _CH, D), jnp.float32),
            pltpu.VMEM((2, _CH, D // 2), jnp.int32),
            pltpu.VMEM((2, _CH, D), jnp.float32),
            pltpu.VMEM_SHARED((N_pad, D), jnp.float32),
            pltpu.SemaphoreType.DMA,
            pltpu.SemaphoreType.DMA,
            pltpu.SemaphoreType.DMA,
            pltpu.SemaphoreType.DMA,
            pltpu.SemaphoreType.DMA,
            pltpu.SemaphoreType.DMA,
            pltpu.SemaphoreType.DMA,
            pltpu.SemaphoreType.DMA,
            pltpu.SemaphoreType.DMA,
            pltpu.SemaphoreType.DMA,
        ],
    )
    def sc_layer(h_hbm, e_hbm, src_hbm, dst_hbm, out_hbm,
                 srcv, dstv, hbuf, ebuf, mbuf, agg_sh,
                 gsem0, gsem1, esem0, esem1, ssem0, ssem1,
                 isem0, isem1, dsem0, dsem1):
        gsem = (gsem0, gsem1)
        esem = (esem0, esem1)
        ssem = (ssem0, ssem1)
        isem = (isem0, isem1)
        dsem = (dsem0, dsem1)
        c = lax.axis_index("c")
        s = lax.axis_index("s")
        nchk = jnp.where(c == 0, C0, C1)
        tb = jnp.where(c == 0, s * C0, _NS * C0 + s * C1)

        def wait_e(p):
            pltpu.make_async_copy(e_hbm.at[pl.ds(0, _CH)], ebuf.at[p],
                                  esem[p]).wait()

        def wait_g(p):
            pltpu.make_async_copy(h_hbm.at[srcv.at[0]], hbuf.at[p],
                                  gsem[p]).wait()

        def wait_s(p):
            pltpu.make_async_copy(mbuf.at[p], agg_sh.at[dstv.at[0]],
                                  ssem[p]).wait()

        def wait_idx(ring, sem):
            pltpu.make_async_copy(src_hbm.at[pl.ds(0, _CH)], ring.at[0],
                                  sem).wait()

        def zrow(r, carry):
            for j in range(D // 16):
                mbuf[0, r, pl.ds(j * 16, 16)] = jnp.zeros((16,), jnp.float32)
            return carry

        lax.fori_loop(0, _CH, zrow, 0)
        for t in range(ZR):
            rows = min(_CH, RPT - t * _CH)
            pltpu.sync_copy(mbuf.at[0, pl.ds(0, rows)],
                            agg_sh.at[pl.ds(s * RPT + t * _CH, rows)])

        for k in range(4):
            pltpu.sync_copy(src_hbm.at[pl.ds((tb + k) * _CH, _CH)],
                            srcv.at[k])
        for k in range(2):
            pltpu.sync_copy(dst_hbm.at[pl.ds((tb + k) * _CH, _CH)],
                            dstv.at[k])
        for k in range(2):
            pltpu.async_copy(e_hbm.at[pl.ds((tb + k) * _CH, _CH)],
                             ebuf.at[k], esem[k])
            pltpu.async_copy(h_hbm.at[srcv.at[k]], hbuf.at[k], gsem[k])
        plsc.subcore_barrier()

        def quad(g, carry):
            for u in range(4):
                k = g * 4 + u
                p = u % 2
                wait_e(p)
                wait_g(p)

                @pl.when(k >= 2)
                def _():
                    wait_s(p)

                @pl.when(k + 2 < nchk)
                def _():
                    pltpu.async_copy(
                        dst_hbm.at[pl.ds((tb + k + 2) * _CH, _CH)],
                        dstv.at[(u + 2) % 4], dsem[[1, 1, 0, 0][u]])

                @pl.when(k + 4 < nchk)
                def _():
                    pltpu.async_copy(
                        src_hbm.at[pl.ds((tb + k + 4) * _CH, _CH)],
                        srcv.at[u], isem[[0, 0, 1, 1][u]])

                def row(r, rc):
                    for rr in range(2):
                        ri = r * 2 + rr
                        for j in range(D // 32):
                            w = ebuf[p, ri, pl.ds(j * 16, 16)]
                            e0 = lax.bitcast_convert_type(
                                w << 16, jnp.float32)
                            e1 = lax.bitcast_convert_type(
                                w & jnp.int32(-65536), jnp.float32)
                            s0 = pl.ds(j * 32, 16)
                            s1 = pl.ds(j * 32 + 16, 16)
                            mbuf[p, ri, s0] = jnp.maximum(
                                hbuf[p, ri, s0] + e0, 0.0)
                            mbuf[p, ri, s1] = jnp.maximum(
                                hbuf[p, ri, s1] + e1, 0.0)
                    return rc

                lax.fori_loop(0, _CH // 2, row, 0)

                @pl.when(k >= 2)
                def _():
                    wait_idx(dstv, dsem[[0, 0, 1, 1][u]])

                pltpu.async_copy(mbuf.at[p], agg_sh.at[dstv.at[u]],
                                 ssem[p], add=True)

                @pl.when(jnp.logical_and(k >= 2, k + 2 < nchk))
                def _():
                    wait_idx(srcv, isem[[1, 1, 0, 0][u]])

                @pl.when(k + 2 < nchk)
                def _():
                    pltpu.async_copy(
                        e_hbm.at[pl.ds((tb + k + 2) * _CH, _CH)],
                        ebuf.at[p], esem[p])
                    pltpu.async_copy(h_hbm.at[srcv.at[(u + 2) % 4]],
                                     hbuf.at[p], gsem[p])
            return carry

        lax.fori_loop(0, nchk // 4, quad, 0)
        for p in range(2):
            wait_s(p)
        plsc.subcore_barrier()
        pltpu.sync_copy(agg_sh.at[pl.ds(s * RPT, RPT)],
                        out_hbm.at[pl.ds(c * N_pad + s * RPT, RPT)])

    return sc_layer


def kernel(x, edge_index, edge_attr, W_in, b_in, W_e, b_e, W_h, b_h):
    N, D = x.shape
    E = edge_index.shape[1]
    depth = W_h.shape[0]

    grain = _NC * _NS * _CH * 4
    E_pad = ((E + grain - 1) // grain) * grain
    pad = E_pad - E
    N_pad = ((N + _NS * 8 - 1) // (_NS * 8)) * (_NS * 8)
    if N_pad == N:
        N_pad += _NS * 8

    src = jnp.concatenate([edge_index[0].astype(jnp.int32),
                           jnp.zeros((pad,), jnp.int32)])
    dst = jnp.concatenate([edge_index[1].astype(jnp.int32),
                           jnp.full((pad,), N, jnp.int32)])

    perm = jnp.array(
        [32 * (m // 16) + m % 16 for m in range(D // 2)]
        + [32 * (m // 16) + 16 + m % 16 for m in range(D // 2)],
        dtype=jnp.int32)
    h = _tc_proj(x, W_in, b_in, blk=2000)
    e = _tc_edge_proj(edge_attr, W_e[:, perm], b_e[perm], E_pad, blk=4096)

    chunks_per_s = E_pad // (_NS * _CH)
    c0 = chunks_per_s - 4
    sc_layer = _make_sc_layer(N, D, E_pad, N_pad, c0, chunks_per_s - c0)
    for i in range(depth):
        agg = sc_layer(h, e, src, dst)
        h = _tc_update(h, agg[:N], agg[N_pad:N_pad + N], W_h[i], b_h[i],
                       blk=2000)
    return h


# speedup vs baseline: 1.2182x; 1.2182x over previous
import functools

import jax
import jax.numpy as jnp
from jax import lax
from jax.experimental import pallas as pl
from jax.experimental.pallas import tpu as pltpu
from jax.experimental.pallas import tpu_sc as plsc

_NC = 2
_NS = 16
_CH = 64


def _tc_proj(x, W, b, blk):
    M, K = x.shape
    Do = W.shape[1]

    def body(x_ref, w_ref, b_ref, o_ref):
        v = jnp.dot(x_ref[...], w_ref[...], preferred_element_type=jnp.float32)
        o_ref[...] = jnp.maximum(v + b_ref[...], 0.0)

    return pl.pallas_call(
        body,
        grid=(M // blk,),
        in_specs=[
            pl.BlockSpec((blk, K), lambda i: (i, 0)),
            pl.BlockSpec((K, Do), lambda i: (0, 0)),
            pl.BlockSpec((1, Do), lambda i: (0, 0)),
        ],
        out_specs=pl.BlockSpec((blk, Do), lambda i: (i, 0)),
        out_shape=jax.ShapeDtypeStruct((M, Do), jnp.float32),
    )(x, W, b.reshape(1, Do))


def _tc_update(h, a0, a1, W, b, blk):
    M, K = h.shape
    Do = W.shape[1]

    def body(h_ref, a0_ref, a1_ref, w_ref, b_ref, o_ref):
        t = h_ref[...] + a0_ref[...] + a1_ref[...]
        v = jnp.dot(t, w_ref[...], preferred_element_type=jnp.float32)
        o_ref[...] = jnp.maximum(v + b_ref[...], 0.0)

    return pl.pallas_call(
        body,
        grid=(M // blk,),
        in_specs=[
            pl.BlockSpec((blk, K), lambda i: (i, 0)),
            pl.BlockSpec((blk, K), lambda i: (i, 0)),
            pl.BlockSpec((blk, K), lambda i: (i, 0)),
            pl.BlockSpec((K, Do), lambda i: (0, 0)),
            pl.BlockSpec((1, Do), lambda i: (0, 0)),
        ],
        out_specs=pl.BlockSpec((blk, Do), lambda i: (i, 0)),
        out_shape=jax.ShapeDtypeStruct((M, Do), jnp.float32),
    )(h, a0, a1, W, b.reshape(1, Do))


def _tc_edge_proj(ea, W, b, E_pad, blk):
    M, K = ea.shape
    Do = W.shape[1]
    last = (M - 1) // blk

    def body(a_ref, w_ref, b_ref, o_ref):
        v = jnp.dot(a_ref[...], w_ref[...], preferred_element_type=jnp.float32)
        v = jnp.maximum(v + b_ref[...], 0.0)
        bits = lax.bitcast_convert_type(v, jnp.int32)
        rnd = bits + jnp.int32(0x7FFF) + ((bits >> 16) & 1)
        lo = (rnd[:, : Do // 2] >> 16) & jnp.int32(0xFFFF)
        hi = rnd[:, Do // 2 :] & jnp.int32(-65536)
        o_ref[...] = lo | hi

    return pl.pallas_call(
        body,
        grid=(E_pad // blk,),
        in_specs=[
            pl.BlockSpec((blk, K), lambda i: (jnp.minimum(i, last), 0)),
            pl.BlockSpec((K, Do), lambda i: (0, 0)),
            pl.BlockSpec((1, Do), lambda i: (0, 0)),
        ],
        out_specs=pl.BlockSpec((blk, Do // 2), lambda i: (i, 0)),
        out_shape=jax.ShapeDtypeStruct((E_pad, Do // 2), jnp.int32),
    )(ea, W, b.reshape(1, Do))


@functools.cache
def _make_sc_layer(N, D, E_pad, N_pad, C0, C1):
    assert (C0 + C1) * _NS * _CH == E_pad and C0 % 4 == 0 and C1 % 4 == 0
    RPT = N_pad // _NS
    ZR = (RPT + _CH - 1) // _CH
    mesh = plsc.VectorSubcoreMesh(core_axis_name="c", subcore_axis_name="s")

    @functools.partial(
        pl.kernel,
        out_type=jax.ShapeDtypeStruct((_NC * N_pad, D), jnp.float32),
        mesh=mesh,
        scratch_types=[
            pltpu.VMEM((4, _CH), jnp.int32),
            pltpu.VMEM((4, _CH), jnp.int32),
            pltpu.VMEM((2, _CH, D), jnp.float32),
            pltpu.VMEM((2, _CH, D // 2), jnp.int32),
            pltpu.VMEM((2, _CH, D), jnp.float32),
            pltpu.VMEM_SHARED((N_pad, D), jnp.float32),
            pltpu.SemaphoreType.DMA,
            pltpu.SemaphoreType.DMA,
            pltpu.SemaphoreType.DMA,
            pltpu.SemaphoreType.DMA,
            pltpu.SemaphoreType.DMA,
            pltpu.SemaphoreType.DMA,
            pltpu.SemaphoreType.DMA,
            pltpu.SemaphoreType.DMA,
            pltpu.SemaphoreType.DMA,
            pltpu.SemaphoreType.DMA,
        ],
    )
    def sc_layer(h_hbm, e_hbm, src_hbm, dst_hbm, out_hbm,
                 srcv, dstv, hbuf, ebuf, mbuf, agg_sh,
                 gsem0, gsem1, esem0, esem1, ssem0, ssem1,
                 isem0, isem1, dsem0, dsem1):
        gsem = (gsem0, gsem1)
        esem = (esem0, esem1)
        ssem = (ssem0, ssem1)
        isem = (isem0, isem1)
        dsem = (dsem0, dsem1)
        c = lax.axis_index("c")
        s = lax.axis_index("s")
        nchk = jnp.where(c == 0, C0, C1)
        tb = jnp.where(c == 0, s * C0, _NS * C0 + s * C1)

        def wait_e(p):
            pltpu.make_async_copy(e_hbm.at[pl.ds(0, _CH)], ebuf.at[p],
                                  esem[p]).wait()

        def wait_g(p):
            pltpu.make_async_copy(h_hbm.at[srcv.at[0]], hbuf.at[p],
                                  gsem[p]).wait()

        def wait_s(p):
            pltpu.make_async_copy(mbuf.at[p], agg_sh.at[dstv.at[0]],
                                  ssem[p]).wait()

        def wait_idx(ring, sem):
            pltpu.make_async_copy(src_hbm.at[pl.ds(0, _CH)], ring.at[0],
                                  sem).wait()

        def zrow(r, carry):
            for j in range(D // 16):
                mbuf[0, r, pl.ds(j * 16, 16)] = jnp.zeros((16,), jnp.float32)
            return carry

        lax.fori_loop(0, _CH, zrow, 0)
        for t in range(ZR):
            rows = min(_CH, RPT - t * _CH)
            pltpu.sync_copy(mbuf.at[0, pl.ds(0, rows)],
                            agg_sh.at[pl.ds(s * RPT + t * _CH, rows)])

        for k in range(4):
            pltpu.sync_copy(src_hbm.at[pl.ds((tb + k) * _CH, _CH)],
                            srcv.at[k])
        for k in range(2):
            pltpu.sync_copy(dst_hbm.at[pl.ds((tb + k) * _CH, _CH)],
                            dstv.at[k])
        for k in range(2):
            pltpu.async_copy(e_hbm.at[pl.ds((tb + k) * _CH, _CH)],
                             ebuf.at[k], esem[k])
            pltpu.async_copy(h_hbm.at[srcv.at[k]], hbuf.at[k], gsem[k])
        plsc.subcore_barrier()

        def quad(g, carry):
            for u in range(4):
                k = g * 4 + u
                p = u % 2
                wait_e(p)
                wait_g(p)

                @pl.when(k >= 2)
                def _():
                    wait_s(p)

                @pl.when(k + 2 < nchk)
                def _():
                    pltpu.async_copy(
                        dst_hbm.at[pl.ds((tb + k + 2) * _CH, _CH)],
                        dstv.at[(u + 2) % 4], dsem[[1, 1, 0, 0][u]])

                @pl.when(k + 4 < nchk)
                def _():
                    pltpu.async_copy(
                        src_hbm.at[pl.ds((tb + k + 4) * _CH, _CH)],
                        srcv.at[u], isem[[0, 0, 1, 1][u]])

                def row(r, rc):
                    for rr in range(2):
                        ri = r * 2 + rr
                        for j in range(D // 32):
                            w = ebuf[p, ri, pl.ds(j * 16, 16)]
                            e0 = lax.bitcast_convert_type(
                                w << 16, jnp.float32)
                            e1 = lax.bitcast_convert_type(
                                w & jnp.int32(-65536), jnp.float32)
                            s0 = pl.ds(j * 32, 16)
                            s1 = pl.ds(j * 32 + 16, 16)
                            mbuf[p, ri, s0] = jnp.maximum(
                                hbuf[p, ri, s0] + e0, 0.0)
                            mbuf[p, ri, s1] = jnp.maximum(
                                hbuf[p, ri, s1] + e1, 0.0)
                    return rc

                lax.fori_loop(0, _CH // 2, row, 0)

                @pl.when(k >= 2)
                def _():
                    wait_idx(dstv, dsem[[0, 0, 1, 1][u]])

                pltpu.async_copy(mbuf.at[p], agg_sh.at[dstv.at[u]],
                                 ssem[p], add=True)

                @pl.when(jnp.logical_and(k >= 2, k + 2 < nchk))
                def _():
                    wait_idx(srcv, isem[[1, 1, 0, 0][u]])

                @pl.when(k + 2 < nchk)
                def _():
                    pltpu.async_copy(
                        e_hbm.at[pl.ds((tb + k + 2) * _CH, _CH)],
                        ebuf.at[p], esem[p])
                    pltpu.async_copy(h_hbm.at[srcv.at[(u + 2) % 4]],
                                     hbuf.at[p], gsem[p])
            return carry

        lax.fori_loop(0, nchk // 4, quad, 0)
        for p in range(2):
            wait_s(p)
        plsc.subcore_barrier()
        pltpu.sync_copy(agg_sh.at[pl.ds(s * RPT, RPT)],
                        out_hbm.at[pl.ds(c * N_pad + s * RPT, RPT)])

    return sc_layer


def kernel(x, edge_index, edge_attr, W_in, b_in, W_e, b_e, W_h, b_h):
    N, D = x.shape
    E = edge_index.shape[1]
    depth = W_h.shape[0]

    grain = _NC * _NS * _CH * 4
    E_pad = ((E + grain - 1) // grain) * grain
    pad = E_pad - E
    N_pad = ((N + _NS * 8 - 1) // (_NS * 8)) * (_NS * 8)
    if N_pad == N:
        N_pad += _NS * 8

    src = jnp.concatenate([edge_index[0].astype(jnp.int32),
                           jnp.zeros((pad,), jnp.int32)])
    dst = jnp.concatenate([edge_index[1].astype(jnp.int32),
                           jnp.full((pad,), N, jnp.int32)])

    perm = jnp.array(
        [32 * (m // 16) + m % 16 for m in range(D // 2)]
        + [32 * (m // 16) + 16 + m % 16 for m in range(D // 2)],
        dtype=jnp.int32)
    h = _tc_proj(x, W_in, b_in, blk=2000)
    e = _tc_edge_proj(edge_attr, W_e[:, perm], b_e[perm], E_pad, blk=4096)

    chunks_per_s = E_pad // (_NS * _CH)
    c0 = (chunks_per_s * 4 // 5) // 4 * 4
    sc_layer = _make_sc_layer(N, D, E_pad, N_pad, c0, chunks_per_s - c0)
    for i in range(depth):
        agg = sc_layer(h, e, src, dst)
        h = _tc_update(h, agg[:N], agg[N_pad:N_pad + N], W_h[i], b_h[i],
                       blk=2000)
    return h


# speedup vs baseline: 1.2349x; 1.0137x over previous
import functools

import jax
import jax.numpy as jnp
from jax import lax
from jax.experimental import pallas as pl
from jax.experimental.pallas import tpu as pltpu
from jax.experimental.pallas import tpu_sc as plsc

_NC = 2
_NS = 16
_CH = 64


def _tc_proj(x, W, b, blk):
    M, K = x.shape
    Do = W.shape[1]

    def body(x_ref, w_ref, b_ref, o_ref):
        v = jnp.dot(x_ref[...], w_ref[...], preferred_element_type=jnp.float32)
        o_ref[...] = jnp.maximum(v + b_ref[...], 0.0)

    return pl.pallas_call(
        body,
        grid=(M // blk,),
        in_specs=[
            pl.BlockSpec((blk, K), lambda i: (i, 0)),
            pl.BlockSpec((K, Do), lambda i: (0, 0)),
            pl.BlockSpec((1, Do), lambda i: (0, 0)),
        ],
        out_specs=pl.BlockSpec((blk, Do), lambda i: (i, 0)),
        out_shape=jax.ShapeDtypeStruct((M, Do), jnp.float32),
    )(x, W, b.reshape(1, Do))


def _tc_update(h, a0, a1, W, b, blk):
    M, K = h.shape
    Do = W.shape[1]

    def body(h_ref, a0_ref, a1_ref, w_ref, b_ref, o_ref):
        t = h_ref[...] + a0_ref[...] + a1_ref[...]
        v = jnp.dot(t, w_ref[...], preferred_element_type=jnp.float32)
        o_ref[...] = jnp.maximum(v + b_ref[...], 0.0)

    return pl.pallas_call(
        body,
        grid=(M // blk,),
        in_specs=[
            pl.BlockSpec((blk, K), lambda i: (i, 0)),
            pl.BlockSpec((blk, K), lambda i: (i, 0)),
            pl.BlockSpec((blk, K), lambda i: (i, 0)),
            pl.BlockSpec((K, Do), lambda i: (0, 0)),
            pl.BlockSpec((1, Do), lambda i: (0, 0)),
        ],
        out_specs=pl.BlockSpec((blk, Do), lambda i: (i, 0)),
        out_shape=jax.ShapeDtypeStruct((M, Do), jnp.float32),
    )(h, a0, a1, W, b.reshape(1, Do))


def _tc_edge_proj(ea, W, b, E_pad, blk):
    M, K = ea.shape
    Do = W.shape[1]
    last = (M - 1) // blk

    def body(a_ref, w_ref, b_ref, o_ref):
        v = jnp.dot(a_ref[...], w_ref[...], preferred_element_type=jnp.float32)
        v = jnp.maximum(v + b_ref[...], 0.0)
        bits = lax.bitcast_convert_type(v, jnp.int32)
        rnd = bits + jnp.int32(0x7FFF) + ((bits >> 16) & 1)
        lo = (rnd[:, : Do // 2] >> 16) & jnp.int32(0xFFFF)
        hi = rnd[:, Do // 2 :] & jnp.int32(-65536)
        o_ref[...] = lo | hi

    return pl.pallas_call(
        body,
        grid=(E_pad // blk,),
        in_specs=[
            pl.BlockSpec((blk, K), lambda i: (jnp.minimum(i, last), 0)),
            pl.BlockSpec((K, Do), lambda i: (0, 0)),
            pl.BlockSpec((1, Do), lambda i: (0, 0)),
        ],
        out_specs=pl.BlockSpec((blk, Do // 2), lambda i: (i, 0)),
        out_shape=jax.ShapeDtypeStruct((E_pad, Do // 2), jnp.int32),
    )(ea, W, b.reshape(1, Do))


@functools.cache
def _make_sc_layer(N, D, E_pad, N_pad, C0, C1):
    assert (C0 + C1) * _NS * _CH == E_pad and C0 % 4 == 0 and C1 % 4 == 0
    RPT = N_pad // _NS
    ZR = (RPT + _CH - 1) // _CH
    mesh = plsc.VectorSubcoreMesh(core_axis_name="c", subcore_axis_name="s")

    @functools.partial(
        pl.kernel,
        out_type=jax.ShapeDtypeStruct((_NC * N_pad, D), jnp.float32),
        mesh=mesh,
        scratch_types=[
            pltpu.VMEM((4, _CH), jnp.int32),
            pltpu.VMEM((4, _CH), jnp.int32),
            pltpu.VMEM((2, _CH, D), jnp.float32),
            pltpu.VMEM((2, _CH, D // 2), jnp.int32),
            pltpu.VMEM((2, _CH, D), jnp.float32),
            pltpu.VMEM_SHARED((N_pad, D), jnp.float32),
            pltpu.SemaphoreType.DMA,
            pltpu.SemaphoreType.DMA,
            pltpu.SemaphoreType.DMA,
            pltpu.SemaphoreType.DMA,
            pltpu.SemaphoreType.DMA,
            pltpu.SemaphoreType.DMA,
            pltpu.SemaphoreType.DMA,
            pltpu.SemaphoreType.DMA,
            pltpu.SemaphoreType.DMA,
            pltpu.SemaphoreType.DMA,
        ],
    )
    def sc_layer(h_hbm, e_hbm, src_hbm, dst_hbm, out_hbm,
                 srcv, dstv, hbuf, ebuf, mbuf, agg_sh,
                 gsem0, gsem1, esem0, esem1, ssem0, ssem1,
                 isem0, isem1, dsem0, dsem1):
        gsem = (gsem0, gsem1)
        esem = (esem0, esem1)
        ssem = (ssem0, ssem1)
        isem = (isem0, isem1)
        dsem = (dsem0, dsem1)
        c = lax.axis_index("c")
        s = lax.axis_index("s")
        nchk = jnp.where(c == 0, C0, C1)
        tb = jnp.where(c == 0, s * C0, _NS * C0 + s * C1)

        def wait_e(p):
            pltpu.make_async_copy(e_hbm.at[pl.ds(0, _CH)], ebuf.at[p],
                                  esem[p]).wait()

        def wait_g(p):
            pltpu.make_async_copy(h_hbm.at[srcv.at[0]], hbuf.at[p],
                                  gsem[p]).wait()

        def wait_s(p):
            pltpu.make_async_copy(mbuf.at[p], agg_sh.at[dstv.at[0]],
                                  ssem[p]).wait()

        def wait_idx(ring, sem):
            pltpu.make_async_copy(src_hbm.at[pl.ds(0, _CH)], ring.at[0],
                                  sem).wait()

        def zrow(r, carry):
            for j in range(D // 16):
                mbuf[0, r, pl.ds(j * 16, 16)] = jnp.zeros((16,), jnp.float32)
            return carry

        lax.fori_loop(0, _CH, zrow, 0)
        for t in range(ZR):
            rows = min(_CH, RPT - t * _CH)
            pltpu.sync_copy(mbuf.at[0, pl.ds(0, rows)],
                            agg_sh.at[pl.ds(s * RPT + t * _CH, rows)])

        for k in range(4):
            pltpu.sync_copy(src_hbm.at[pl.ds((tb + k) * _CH, _CH)],
                            srcv.at[k])
        for k in range(2):
            pltpu.sync_copy(dst_hbm.at[pl.ds((tb + k) * _CH, _CH)],
                            dstv.at[k])
        for k in range(2):
            pltpu.async_copy(e_hbm.at[pl.ds((tb + k) * _CH, _CH)],
                             ebuf.at[k], esem[k])
            pltpu.async_copy(h_hbm.at[srcv.at[k]], hbuf.at[k], gsem[k])
        plsc.subcore_barrier()

        def quad(g, carry):
            for u in range(4):
                k = g * 4 + u
                p = u % 2
                wait_e(p)
                wait_g(p)

                @pl.when(k >= 2)
                def _():
                    wait_s(p)

                @pl.when(k + 2 < nchk)
                def _():
                    pltpu.async_copy(
                        dst_hbm.at[pl.ds((tb + k + 2) * _CH, _CH)],
                        dstv.at[(u + 2) % 4], dsem[[1, 1, 0, 0][u]])

                @pl.when(k + 4 < nchk)
                def _():
                    pltpu.async_copy(
                        src_hbm.at[pl.ds((tb + k + 4) * _CH, _CH)],
                        srcv.at[u], isem[[0, 0, 1, 1][u]])

                def row(r, rc):
                    for rr in range(2):
                        ri = r * 2 + rr
                        for j in range(D // 32):
                            w = ebuf[p, ri, pl.ds(j * 16, 16)]
                            e0 = lax.bitcast_convert_type(
                                w << 16, jnp.float32)
                            e1 = lax.bitcast_convert_type(
                                w & jnp.int32(-65536), jnp.float32)
                            s0 = pl.ds(j * 32, 16)
                            s1 = pl.ds(j * 32 + 16, 16)
                            mbuf[p, ri, s0] = jnp.maximum(
                                hbuf[p, ri, s0] + e0, 0.0)
                            mbuf[p, ri, s1] = jnp.maximum(
                                hbuf[p, ri, s1] + e1, 0.0)
                    return rc

                lax.fori_loop(0, _CH // 2, row, 0)

                @pl.when(k >= 2)
                def _():
                    wait_idx(dstv, dsem[[0, 0, 1, 1][u]])

                pltpu.async_copy(mbuf.at[p], agg_sh.at[dstv.at[u]],
                                 ssem[p], add=True)

                @pl.when(jnp.logical_and(k >= 2, k + 2 < nchk))
                def _():
                    wait_idx(srcv, isem[[1, 1, 0, 0][u]])

                @pl.when(k + 2 < nchk)
                def _():
                    pltpu.async_copy(
                        e_hbm.at[pl.ds((tb + k + 2) * _CH, _CH)],
                        ebuf.at[p], esem[p])
                    pltpu.async_copy(h_hbm.at[srcv.at[(u + 2) % 4]],
                                     hbuf.at[p], gsem[p])
            return carry

        lax.fori_loop(0, nchk // 4, quad, 0)
        for p in range(2):
            wait_s(p)
        plsc.subcore_barrier()
        pltpu.sync_copy(agg_sh.at[pl.ds(s * RPT, RPT)],
                        out_hbm.at[pl.ds(c * N_pad + s * RPT, RPT)])

    return sc_layer


def kernel(x, edge_index, edge_attr, W_in, b_in, W_e, b_e, W_h, b_h):
    N, D = x.shape
    E = edge_index.shape[1]
    depth = W_h.shape[0]

    grain = _NC * _NS * _CH * 4
    E_pad = ((E + grain - 1) // grain) * grain
    pad = E_pad - E
    N_pad = ((N + _NS * 8 - 1) // (_NS * 8)) * (_NS * 8)
    if N_pad == N:
        N_pad += _NS * 8

    src = jnp.concatenate([edge_index[0].astype(jnp.int32),
                           jnp.zeros((pad,), jnp.int32)])
    dst = jnp.concatenate([edge_index[1].astype(jnp.int32),
                           jnp.full((pad,), N, jnp.int32)])

    perm = jnp.array(
        [32 * (m // 16) + m % 16 for m in range(D // 2)]
        + [32 * (m // 16) + 16 + m % 16 for m in range(D // 2)],
        dtype=jnp.int32)
    h = _tc_proj(x, W_in, b_in, blk=2000)
    e = _tc_edge_proj(edge_attr, W_e[:, perm], b_e[perm], E_pad, blk=4096)

    chunks_per_s = E_pad // (_NS * _CH)
    c0 = (chunks_per_s * 17 // 20) // 4 * 4
    sc_layer = _make_sc_layer(N, D, E_pad, N_pad, c0, chunks_per_s - c0)
    for i in range(depth):
        agg = sc_layer(h, e, src, dst)
        h = _tc_update(h, agg[:N], agg[N_pad:N_pad + N], W_h[i], b_h[i],
                       blk=2000)
    return h


# speedup vs baseline: 1.2781x; 1.0350x over previous
import functools

import jax
import jax.numpy as jnp
from jax import lax
from jax.experimental import pallas as pl
from jax.experimental.pallas import tpu as pltpu
from jax.experimental.pallas import tpu_sc as plsc

_NC = 2
_NS = 16
_CH = 64


def _tc_proj(x, W, b, blk):
    M, K = x.shape
    Do = W.shape[1]

    def body(x_ref, w_ref, b_ref, o_ref):
        v = jnp.dot(x_ref[...], w_ref[...], preferred_element_type=jnp.float32)
        o_ref[...] = jnp.maximum(v + b_ref[...], 0.0)

    return pl.pallas_call(
        body,
        grid=(M // blk,),
        in_specs=[
            pl.BlockSpec((blk, K), lambda i: (i, 0)),
            pl.BlockSpec((K, Do), lambda i: (0, 0)),
            pl.BlockSpec((1, Do), lambda i: (0, 0)),
        ],
        out_specs=pl.BlockSpec((blk, Do), lambda i: (i, 0)),
        out_shape=jax.ShapeDtypeStruct((M, Do), jnp.float32),
    )(x, W, b.reshape(1, Do))


def _tc_update(h, a0, a1, W, b, blk):
    M, K = h.shape
    Do = W.shape[1]

    def body(h_ref, a0_ref, a1_ref, w_ref, b_ref, o_ref):
        t = h_ref[...] + a0_ref[...] + a1_ref[...]
        v = jnp.dot(t, w_ref[...], preferred_element_type=jnp.float32)
        o_ref[...] = jnp.maximum(v + b_ref[...], 0.0)

    return pl.pallas_call(
        body,
        grid=(M // blk,),
        in_specs=[
            pl.BlockSpec((blk, K), lambda i: (i, 0)),
            pl.BlockSpec((blk, K), lambda i: (i, 0)),
            pl.BlockSpec((blk, K), lambda i: (i, 0)),
            pl.BlockSpec((K, Do), lambda i: (0, 0)),
            pl.BlockSpec((1, Do), lambda i: (0, 0)),
        ],
        out_specs=pl.BlockSpec((blk, Do), lambda i: (i, 0)),
        out_shape=jax.ShapeDtypeStruct((M, Do), jnp.float32),
    )(h, a0, a1, W, b.reshape(1, Do))


def _tc_edge_proj(ea, W, b, E_pad, blk):
    M, K = ea.shape
    Do = W.shape[1]
    last = (M - 1) // blk

    def body(a_ref, w_ref, b_ref, o_ref):
        v = jnp.dot(a_ref[...], w_ref[...], preferred_element_type=jnp.float32)
        v = jnp.maximum(v + b_ref[...], 0.0)
        bits = lax.bitcast_convert_type(v, jnp.int32)
        rnd = bits + jnp.int32(0x7FFF) + ((bits >> 16) & 1)
        lo = (rnd[:, : Do // 2] >> 16) & jnp.int32(0xFFFF)
        hi = rnd[:, Do // 2 :] & jnp.int32(-65536)
        o_ref[...] = lo | hi

    return pl.pallas_call(
        body,
        grid=(E_pad // blk,),
        in_specs=[
            pl.BlockSpec((blk, K), lambda i: (jnp.minimum(i, last), 0)),
            pl.BlockSpec((K, Do), lambda i: (0, 0)),
            pl.BlockSpec((1, Do), lambda i: (0, 0)),
        ],
        out_specs=pl.BlockSpec((blk, Do // 2), lambda i: (i, 0)),
        out_shape=jax.ShapeDtypeStruct((E_pad, Do // 2), jnp.int32),
    )(ea, W, b.reshape(1, Do))


@functools.cache
def _make_sc_layer(N, D, E_pad, N_pad, C0, C1):
    assert (C0 + C1) * _NS * _CH == E_pad and C0 % 4 == 0 and C1 % 4 == 0
    RPT = N_pad // _NS
    ZR = (RPT + _CH - 1) // _CH
    mesh = plsc.VectorSubcoreMesh(core_axis_name="c", subcore_axis_name="s")

    @functools.partial(
        pl.kernel,
        out_type=jax.ShapeDtypeStruct((_NC * N_pad, D), jnp.float32),
        mesh=mesh,
        scratch_types=[
            pltpu.VMEM((4, _CH), jnp.int32),
            pltpu.VMEM((4, _CH), jnp.int32),
            pltpu.VMEM((2, _CH, D), jnp.float32),
            pltpu.VMEM((2, _CH, D // 2), jnp.int32),
            pltpu.VMEM((2, _CH, D), jnp.float32),
            pltpu.VMEM_SHARED((N_pad, D), jnp.float32),
            pltpu.SemaphoreType.DMA,
            pltpu.SemaphoreType.DMA,
            pltpu.SemaphoreType.DMA,
            pltpu.SemaphoreType.DMA,
            pltpu.SemaphoreType.DMA,
            pltpu.SemaphoreType.DMA,
            pltpu.SemaphoreType.DMA,
            pltpu.SemaphoreType.DMA,
            pltpu.SemaphoreType.DMA,
            pltpu.SemaphoreType.DMA,
        ],
    )
    def sc_layer(h_hbm, e_hbm, src_hbm, dst_hbm, out_hbm,
                 srcv, dstv, hbuf, ebuf, mbuf, agg_sh,
                 gsem0, gsem1, esem0, esem1, ssem0, ssem1,
                 isem0, isem1, dsem0, dsem1):
        gsem = (gsem0, gsem1)
        esem = (esem0, esem1)
        ssem = (ssem0, ssem1)
        isem = (isem0, isem1)
        dsem = (dsem0, dsem1)
        c = lax.axis_index("c")
        s = lax.axis_index("s")
        nchk = jnp.where(c == 0, C0, C1)
        tb = jnp.where(c == 0, s * C0, _NS * C0 + s * C1)

        def wait_e(p):
            pltpu.make_async_copy(e_hbm.at[pl.ds(0, _CH)], ebuf.at[p],
                                  esem[p]).wait()

        def wait_g(p):
            pltpu.make_async_copy(h_hbm.at[srcv.at[0]], hbuf.at[p],
                                  gsem[p]).wait()

        def wait_s(p):
            pltpu.make_async_copy(mbuf.at[p], agg_sh.at[dstv.at[0]],
                                  ssem[p]).wait()

        def wait_idx(ring, sem):
            pltpu.make_async_copy(src_hbm.at[pl.ds(0, _CH)], ring.at[0],
                                  sem).wait()

        def zrow(r, carry):
            for j in range(D // 16):
                mbuf[0, r, pl.ds(j * 16, 16)] = jnp.zeros((16,), jnp.float32)
            return carry

        lax.fori_loop(0, _CH, zrow, 0)
        for t in range(ZR):
            rows = min(_CH, RPT - t * _CH)
            pltpu.sync_copy(mbuf.at[0, pl.ds(0, rows)],
                            agg_sh.at[pl.ds(s * RPT + t * _CH, rows)])

        for k in range(4):
            pltpu.sync_copy(src_hbm.at[pl.ds((tb + k) * _CH, _CH)],
                            srcv.at[k])
        for k in range(2):
            pltpu.sync_copy(dst_hbm.at[pl.ds((tb + k) * _CH, _CH)],
                            dstv.at[k])
        for k in range(2):
            pltpu.async_copy(e_hbm.at[pl.ds((tb + k) * _CH, _CH)],
                             ebuf.at[k], esem[k])
            pltpu.async_copy(h_hbm.at[srcv.at[k]], hbuf.at[k], gsem[k])
        plsc.subcore_barrier()

        def quad(g, carry):
            for u in range(4):
                k = g * 4 + u
                p = u % 2
                wait_e(p)
                wait_g(p)

                @pl.when(k >= 2)
                def _():
                    wait_s(p)

                @pl.when(k + 2 < nchk)
                def _():
                    pltpu.async_copy(
                        dst_hbm.at[pl.ds((tb + k + 2) * _CH, _CH)],
                        dstv.at[(u + 2) % 4], dsem[[1, 1, 0, 0][u]])

                @pl.when(k + 4 < nchk)
                def _():
                    pltpu.async_copy(
                        src_hbm.at[pl.ds((tb + k + 4) * _CH, _CH)],
                        srcv.at[u], isem[[0, 0, 1, 1][u]])

                def row(r, rc):
                    for rr in range(2):
                        ri = r * 2 + rr
                        for j in range(D // 32):
                            w = ebuf[p, ri, pl.ds(j * 16, 16)]
                            e0 = lax.bitcast_convert_type(
                                w << 16, jnp.float32)
                            e1 = lax.bitcast_convert_type(
                                w & jnp.int32(-65536), jnp.float32)
                            s0 = pl.ds(j * 32, 16)
                            s1 = pl.ds(j * 32 + 16, 16)
                            mbuf[p, ri, s0] = jnp.maximum(
                                hbuf[p, ri, s0] + e0, 0.0)
                            mbuf[p, ri, s1] = jnp.maximum(
                                hbuf[p, ri, s1] + e1, 0.0)
                    return rc

                lax.fori_loop(0, _CH // 2, row, 0)

                @pl.when(k >= 2)
                def _():
                    wait_idx(dstv, dsem[[0, 0, 1, 1][u]])

                pltpu.async_copy(mbuf.at[p], agg_sh.at[dstv.at[u]],
                                 ssem[p], add=True)

                @pl.when(jnp.logical_and(k >= 2, k + 2 < nchk))
                def _():
                    wait_idx(srcv, isem[[1, 1, 0, 0][u]])

                @pl.when(k + 2 < nchk)
                def _():
                    pltpu.async_copy(
                        e_hbm.at[pl.ds((tb + k + 2) * _CH, _CH)],
                        ebuf.at[p], esem[p])
                    pltpu.async_copy(h_hbm.at[srcv.at[(u + 2) % 4]],
                                     hbuf.at[p], gsem[p])
            return carry

        lax.fori_loop(0, nchk // 4, quad, 0)
        for p in range(2):
            wait_s(p)
        plsc.subcore_barrier()
        pltpu.sync_copy(agg_sh.at[pl.ds(s * RPT, RPT)],
                        out_hbm.at[pl.ds(c * N_pad + s * RPT, RPT)])

    return sc_layer


def kernel(x, edge_index, edge_attr, W_in, b_in, W_e, b_e, W_h, b_h):
    N, D = x.shape
    E = edge_index.shape[1]
    depth = W_h.shape[0]

    grain = _NC * _NS * _CH * 4
    E_pad = ((E + grain - 1) // grain) * grain
    pad = E_pad - E
    N_pad = ((N + _NS * 8 - 1) // (_NS * 8)) * (_NS * 8)
    if N_pad == N:
        N_pad += _NS * 8

    src = jnp.concatenate([edge_index[0].astype(jnp.int32),
                           jnp.zeros((pad,), jnp.int32)])
    dst = jnp.concatenate([edge_index[1].astype(jnp.int32),
                           jnp.full((pad,), N, jnp.int32)])

    perm = jnp.array(
        [32 * (m // 16) + m % 16 for m in range(D // 2)]
        + [32 * (m // 16) + 16 + m % 16 for m in range(D // 2)],
        dtype=jnp.int32)
    h = _tc_proj(x, W_in, b_in, blk=2000)
    e = _tc_edge_proj(edge_attr, W_e[:, perm], b_e[perm], E_pad, blk=4096)

    chunks_per_s = E_pad // (_NS * _CH)
    c0 = (chunks_per_s * 9 // 10) // 4 * 4
    sc_layer = _make_sc_layer(N, D, E_pad, N_pad, c0, chunks_per_s - c0)
    for i in range(depth):
        agg = sc_layer(h, e, src, dst)
        h = _tc_update(h, agg[:N], agg[N_pad:N_pad + N], W_h[i], b_h[i],
                       blk=2000)
    return h
